# baseline (speedup 1.0000x reference)
import jax, jax.numpy as jnp
from jax.experimental import pallas as pl
from jax.experimental.pallas import tpu as pltpu

_N = 8
_ROWS = 1024

def _body(x_ref, o_ref, buf, sems):
    cs = [pltpu.make_async_copy(
            x_ref.at[pl.ds(i*_ROWS, _ROWS)], buf.at[i], sems.at[i])
          for i in range(_N)]
    for c in cs: c.start()
    for c in cs: c.wait()
    o_ref[...] = buf[0, :8, :128]

def kernel(x):
    flat = x.reshape(12288, 1024)
    out = pl.pallas_call(
        _body,
        in_specs=[pl.BlockSpec(memory_space=pltpu.MemorySpace.HBM)],
        out_specs=pl.BlockSpec(memory_space=pltpu.MemorySpace.VMEM),
        out_shape=jax.ShapeDtypeStruct((8, 128), jnp.float32),
        scratch_shapes=[pltpu.VMEM((_N, _ROWS, 1024), jnp.float32),
                        pltpu.SemaphoreType.DMA((_N,))],
    )(flat)
    return out
